# baseline (speedup 1.0000x reference)
import functools

import jax
import jax.numpy as jnp
from jax import lax
from jax.experimental import pallas as pl
from jax.experimental.pallas import tpu as pltpu
from jax.experimental.pallas import tpu_sc as plsc

N = 10000
E = 320000
F_IN = 128
HID = 128
CLS = 16

NC = 2
NS = 16
NW = NC * NS
EPW = E // NW
RPT = N // NS
ZR = 25

BH = 125
NCH = EPW // BH
HALF = NCH // 2
BC = 2500
NCC = EPW // BC

_MESH = plsc.VectorSubcoreMesh(core_axis_name="c", subcore_axis_name="s")
_SC_PARAMS = pltpu.CompilerParams(use_tc_tiling_on_sc=False)


def _zero_fill(buf, nrows, ncol):
    def body(i, _):
        for k in range(ncol // 16):
            buf[i, pl.ds(k * 16, 16)] = jnp.zeros((16,), jnp.float32)
        return 0
    lax.fori_loop(0, nrows, body, 0)


@functools.partial(
    pl.kernel,
    out_type=jax.ShapeDtypeStruct((NC, N, 16), jnp.float32),
    mesh=_MESH,
    compiler_params=_SC_PARAMS,
    scratch_types=[
        pltpu.VMEM_SHARED((N, 16), jnp.float32),
        pltpu.VMEM((NCC, BC), jnp.int32),
        pltpu.VMEM((BC, 16), jnp.float32),
        pltpu.VMEM((RPT, 16), jnp.float32),
        pltpu.SemaphoreType.DMA,
    ],
)
def _deg_kernel(dst_hbm, out_hbm, acc, dstv, ones, zbuf, sem):
    c = lax.axis_index("c")
    s = lax.axis_index("s")
    wid = c * NS + s

    def fill_ones(i, _):
        ones[i, :] = jnp.ones((16,), jnp.float32)
        return 0
    lax.fori_loop(0, BC, fill_ones, 0)
    _zero_fill(zbuf, RPT, 16)

    pltpu.sync_copy(dst_hbm.at[wid], dstv)
    pltpu.sync_copy(zbuf, acc.at[pl.ds(s * RPT, RPT)])
    plsc.subcore_barrier()

    def chunk(i, _):
        pltpu.async_copy(ones, acc.at[dstv.at[i]], sem, add=True)
        return 0
    lax.fori_loop(0, NCC, chunk, 0)
    def drain(i, _):
        pltpu.make_async_copy(ones, acc.at[dstv.at[i]], sem).wait()
        return 0
    lax.fori_loop(0, NCC, drain, 0)
    plsc.subcore_barrier()

    pltpu.sync_copy(acc.at[pl.ds(s * RPT, RPT)],
                    out_hbm.at[c].at[pl.ds(s * RPT, RPT)])


def _make_prop(F, CB, CN, PH):
    CPP = CN // PH

    @functools.partial(
        pl.kernel,
        out_type=jax.ShapeDtypeStruct((NC, N, F), jnp.float32),
        mesh=_MESH,
        compiler_params=_SC_PARAMS,
        scratch_types=[
            pltpu.VMEM_SHARED((N, F), jnp.float32),
            pltpu.VMEM((CPP, CB), jnp.int32),
            pltpu.VMEM((CPP, CB), jnp.int32),
            pltpu.VMEM((CB, F), jnp.float32),
            pltpu.VMEM((CB, F), jnp.float32),
            pltpu.VMEM((ZR, F), jnp.float32),
            pltpu.SemaphoreType.DMA,
            pltpu.SemaphoreType.DMA,
            pltpu.SemaphoreType.DMA,
            pltpu.SemaphoreType.DMA,
        ],
    )
    def _prop(src_hbm, dst_hbm, hs_hbm, out_hbm, acc, srcv, dstv,
              rows0, rows1, zbuf, semg0, semg1, sems0, sems1):
        c = lax.axis_index("c")
        s = lax.axis_index("s")
        wid = c * NS + s

        _zero_fill(zbuf, ZR, F)
        def zinit(r, _):
            pltpu.async_copy(zbuf, acc.at[pl.ds(s * RPT + r * ZR, ZR)], semg0)
            return 0
        lax.fori_loop(0, RPT // ZR, zinit, 0)
        def zdrain(r, _):
            pltpu.make_async_copy(
                zbuf, acc.at[pl.ds(s * RPT + r * ZR, ZR)], semg0).wait()
            return 0
        lax.fori_loop(0, RPT // ZR, zdrain, 0)

        for ph in range(PH):
            if PH == 1:
                pltpu.sync_copy(src_hbm.at[wid], srcv)
                pltpu.sync_copy(dst_hbm.at[wid], dstv)
            else:
                pltpu.sync_copy(src_hbm.at[wid].at[pl.ds(ph * CPP, CPP)], srcv)
                pltpu.sync_copy(dst_hbm.at[wid].at[pl.ds(ph * CPP, CPP)], dstv)
            if ph == 0:
                plsc.subcore_barrier()

            pltpu.async_copy(hs_hbm.at[srcv.at[0]], rows0, semg0)

            def pair(p, _):
                i = 2 * p
                pltpu.make_async_copy(hs_hbm.at[srcv.at[i]], rows0,
                                      semg0).wait()
                pltpu.async_copy(rows0, acc.at[dstv.at[i]], sems0, add=True)
                pltpu.async_copy(hs_hbm.at[srcv.at[i + 1]], rows1, semg1)
                pltpu.make_async_copy(hs_hbm.at[srcv.at[i + 1]], rows1,
                                      semg1).wait()
                pltpu.make_async_copy(rows0, acc.at[dstv.at[i]], sems0).wait()
                pltpu.async_copy(rows1, acc.at[dstv.at[i + 1]], sems1,
                                 add=True)

                @pl.when(i + 2 < CPP)
                def _():
                    pltpu.async_copy(hs_hbm.at[srcv.at[i + 2]], rows0, semg0)
                pltpu.make_async_copy(rows1, acc.at[dstv.at[i + 1]],
                                      sems1).wait()
                return 0
            lax.fori_loop(0, CPP // 2, pair, 0)

        plsc.subcore_barrier()
        pltpu.sync_copy(acc.at[pl.ds(s * RPT, RPT)],
                        out_hbm.at[c].at[pl.ds(s * RPT, RPT)])

    return _prop


_prop_hid = _make_prop(HID, BH, NCH, 2)
_prop_cls = _make_prop(CLS, BC, NCC, 1)


def _tc1_body(degp_ref, x_ref, w1_ref, dinv_ref, hs1_ref):
    deg = (jnp.max(degp_ref[0], axis=1, keepdims=True)
           + jnp.max(degp_ref[1], axis=1, keepdims=True) + 1.0)
    dinv = lax.rsqrt(deg)
    dinv_ref[...] = dinv
    h = jnp.dot(x_ref[...], w1_ref[...], preferred_element_type=jnp.float32)
    hs1_ref[...] = h * dinv


def _tc2_body(accp_ref, hs1_ref, dinv_ref, w2_ref, b1_ref, hs2_ref):
    dinv = dinv_ref[...]
    out1 = dinv * (accp_ref[0] + accp_ref[1] + hs1_ref[...]) + b1_ref[...]
    z = jnp.maximum(out1, 0.0)
    h2 = jnp.dot(z, w2_ref[...], preferred_element_type=jnp.float32)
    hs2_ref[...] = h2 * dinv


def _tc3_body(accp_ref, hs2_ref, dinv_ref, b2_ref, out_ref):
    logits = (dinv_ref[...] * (accp_ref[0] + accp_ref[1] + hs2_ref[...])
              + b2_ref[...])
    m = jnp.max(logits, axis=1, keepdims=True)
    lse = jnp.log(jnp.sum(jnp.exp(logits - m), axis=1, keepdims=True))
    out_ref[...] = logits - m - lse


_G = 5
_BN = N // _G

_tc1 = pl.pallas_call(
    _tc1_body,
    grid=(_G,),
    in_specs=[
        pl.BlockSpec((NC, _BN, 16), lambda i: (0, i, 0)),
        pl.BlockSpec((_BN, F_IN), lambda i: (i, 0)),
        pl.BlockSpec((F_IN, HID), lambda i: (0, 0)),
    ],
    out_specs=[
        pl.BlockSpec((_BN, 1), lambda i: (i, 0)),
        pl.BlockSpec((_BN, HID), lambda i: (i, 0)),
    ],
    out_shape=[
        jax.ShapeDtypeStruct((N, 1), jnp.float32),
        jax.ShapeDtypeStruct((N, HID), jnp.float32),
    ],
)

_tc2 = pl.pallas_call(
    _tc2_body,
    grid=(_G,),
    in_specs=[
        pl.BlockSpec((NC, _BN, HID), lambda i: (0, i, 0)),
        pl.BlockSpec((_BN, HID), lambda i: (i, 0)),
        pl.BlockSpec((_BN, 1), lambda i: (i, 0)),
        pl.BlockSpec((HID, CLS), lambda i: (0, 0)),
        pl.BlockSpec((1, HID), lambda i: (0, 0)),
    ],
    out_specs=pl.BlockSpec((_BN, CLS), lambda i: (i, 0)),
    out_shape=jax.ShapeDtypeStruct((N, CLS), jnp.float32),
)

_tc3 = pl.pallas_call(
    _tc3_body,
    grid=(_G,),
    in_specs=[
        pl.BlockSpec((NC, _BN, CLS), lambda i: (0, i, 0)),
        pl.BlockSpec((_BN, CLS), lambda i: (i, 0)),
        pl.BlockSpec((_BN, 1), lambda i: (i, 0)),
        pl.BlockSpec((1, CLS), lambda i: (0, 0)),
    ],
    out_specs=pl.BlockSpec((_BN, CLS), lambda i: (i, 0)),
    out_shape=jax.ShapeDtypeStruct((N, CLS), jnp.float32),
)


@jax.jit
def kernel(x, edge_index, W1, b1, W2, b2):
    srch = edge_index[0].reshape(NW, NCH, BH)
    dsth = edge_index[1].reshape(NW, NCH, BH)
    srcc = edge_index[0].reshape(NW, NCC, BC)
    dstc = edge_index[1].reshape(NW, NCC, BC)
    b1r = b1.reshape(1, HID)
    b2r = b2.reshape(1, CLS)

    degp = _deg_kernel(dstc)
    dinv, hs1 = _tc1(degp, x, W1)
    acc1 = _prop_hid(srch, dsth, hs1)
    hs2 = _tc2(acc1, hs1, dinv, W2, b1r)
    acc2 = _prop_cls(srcc, dstc, hs2)
    return _tc3(acc2, hs2, dinv, b2r)


# speedup vs baseline: 1.0404x; 1.0404x over previous
import functools

import jax
import jax.numpy as jnp
from jax import lax
from jax.experimental import pallas as pl
from jax.experimental.pallas import tpu as pltpu
from jax.experimental.pallas import tpu_sc as plsc

N = 10000
E = 320000
F_IN = 128
HID = 128
CLS = 16

NC = 2
NS = 16
NW = NC * NS
EPW = E // NW
RPT = N // NS
ZR = 25

BH = 125
NCH = EPW // BH
HALF = NCH // 2
BC = 2500
NCC = EPW // BC

_MESH = plsc.VectorSubcoreMesh(core_axis_name="c", subcore_axis_name="s")
_SC_PARAMS = pltpu.CompilerParams(use_tc_tiling_on_sc=False)


def _zero_fill(buf, nrows, ncol):
    def body(i, _):
        for k in range(ncol // 16):
            buf[i, pl.ds(k * 16, 16)] = jnp.zeros((16,), jnp.float32)
        return 0
    lax.fori_loop(0, nrows, body, 0)


@functools.partial(
    pl.kernel,
    out_type=jax.ShapeDtypeStruct((NC, N, 16), jnp.float32),
    mesh=_MESH,
    compiler_params=_SC_PARAMS,
    scratch_types=[
        pltpu.VMEM_SHARED((N, 16), jnp.float32),
        pltpu.VMEM((NCC, BC), jnp.int32),
        pltpu.VMEM((BC, 16), jnp.float32),
        pltpu.VMEM((RPT, 16), jnp.float32),
        pltpu.SemaphoreType.DMA,
    ],
)
def _deg_kernel(dst_hbm, out_hbm, acc, dstv, ones, zbuf, sem):
    c = lax.axis_index("c")
    s = lax.axis_index("s")
    wid = c * NS + s

    def fill_ones(i, _):
        for k in range(4):
            ones[i * 4 + k, :] = jnp.ones((16,), jnp.float32)
        return 0
    lax.fori_loop(0, BC // 4, fill_ones, 0)
    def fill_zero(i, _):
        for k in range(5):
            zbuf[i * 5 + k, :] = jnp.zeros((16,), jnp.float32)
        return 0
    lax.fori_loop(0, RPT // 5, fill_zero, 0)

    pltpu.sync_copy(dst_hbm.at[wid], dstv)
    pltpu.sync_copy(zbuf, acc.at[pl.ds(s * RPT, RPT)])
    plsc.subcore_barrier()

    def chunk(i, _):
        pltpu.async_copy(ones, acc.at[dstv.at[i]], sem, add=True)
        return 0
    lax.fori_loop(0, NCC, chunk, 0)
    def drain(i, _):
        pltpu.make_async_copy(ones, acc.at[dstv.at[i]], sem).wait()
        return 0
    lax.fori_loop(0, NCC, drain, 0)
    plsc.subcore_barrier()

    pltpu.sync_copy(acc.at[pl.ds(s * RPT, RPT)],
                    out_hbm.at[c].at[pl.ds(s * RPT, RPT)])


def _make_prop(F, CB, CN, PH):
    CPP = CN // PH

    @functools.partial(
        pl.kernel,
        out_type=jax.ShapeDtypeStruct((NC, N, F), jnp.float32),
        mesh=_MESH,
        compiler_params=_SC_PARAMS,
        scratch_types=[
            pltpu.VMEM_SHARED((N, F), jnp.float32),
            pltpu.VMEM((CPP, CB), jnp.int32),
            pltpu.VMEM((CPP, CB), jnp.int32),
            pltpu.VMEM((CB, F), jnp.float32),
            pltpu.VMEM((CB, F), jnp.float32),
            pltpu.VMEM((ZR, F), jnp.float32),
            pltpu.SemaphoreType.DMA,
            pltpu.SemaphoreType.DMA,
            pltpu.SemaphoreType.DMA,
            pltpu.SemaphoreType.DMA,
        ],
    )
    def _prop(src_hbm, dst_hbm, hs_hbm, out_hbm, acc, srcv, dstv,
              rows0, rows1, zbuf, semg0, semg1, sems0, sems1):
        c = lax.axis_index("c")
        s = lax.axis_index("s")
        wid = c * NS + s

        _zero_fill(zbuf, ZR, F)
        def zinit(r, _):
            pltpu.async_copy(zbuf, acc.at[pl.ds(s * RPT + r * ZR, ZR)], semg0)
            return 0
        lax.fori_loop(0, RPT // ZR, zinit, 0)
        def zdrain(r, _):
            pltpu.make_async_copy(
                zbuf, acc.at[pl.ds(s * RPT + r * ZR, ZR)], semg0).wait()
            return 0
        lax.fori_loop(0, RPT // ZR, zdrain, 0)

        for ph in range(PH):
            if PH == 1:
                pltpu.sync_copy(src_hbm.at[wid], srcv)
                pltpu.sync_copy(dst_hbm.at[wid], dstv)
            else:
                pltpu.sync_copy(src_hbm.at[wid].at[pl.ds(ph * CPP, CPP)], srcv)
                pltpu.sync_copy(dst_hbm.at[wid].at[pl.ds(ph * CPP, CPP)], dstv)
            if ph == 0:
                plsc.subcore_barrier()

            pltpu.async_copy(hs_hbm.at[srcv.at[0]], rows0, semg0)

            def pair(p, _):
                i = 2 * p
                pltpu.make_async_copy(hs_hbm.at[srcv.at[i]], rows0,
                                      semg0).wait()
                pltpu.async_copy(rows0, acc.at[dstv.at[i]], sems0, add=True)
                pltpu.async_copy(hs_hbm.at[srcv.at[i + 1]], rows1, semg1)
                pltpu.make_async_copy(hs_hbm.at[srcv.at[i + 1]], rows1,
                                      semg1).wait()
                pltpu.make_async_copy(rows0, acc.at[dstv.at[i]], sems0).wait()
                pltpu.async_copy(rows1, acc.at[dstv.at[i + 1]], sems1,
                                 add=True)

                @pl.when(i + 2 < CPP)
                def _():
                    pltpu.async_copy(hs_hbm.at[srcv.at[i + 2]], rows0, semg0)
                pltpu.make_async_copy(rows1, acc.at[dstv.at[i + 1]],
                                      sems1).wait()
                return 0
            lax.fori_loop(0, CPP // 2, pair, 0)

        plsc.subcore_barrier()
        pltpu.sync_copy(acc.at[pl.ds(s * RPT, RPT)],
                        out_hbm.at[c].at[pl.ds(s * RPT, RPT)])

    return _prop


_prop_hid = _make_prop(HID, BH, NCH, 2)
_prop_cls = _make_prop(CLS, BC, NCC, 1)


def _tc1_body(degp_ref, x_ref, w1_ref, dinv_ref, hs1_ref):
    deg = (jnp.max(degp_ref[0], axis=1, keepdims=True)
           + jnp.max(degp_ref[1], axis=1, keepdims=True) + 1.0)
    dinv = lax.rsqrt(deg)
    dinv_ref[...] = dinv
    h = jnp.dot(x_ref[...], w1_ref[...], preferred_element_type=jnp.float32)
    hs1_ref[...] = h * dinv


def _tc2_body(accp_ref, hs1_ref, dinv_ref, w2_ref, b1_ref, hs2_ref):
    dinv = dinv_ref[...]
    out1 = dinv * (accp_ref[0] + accp_ref[1] + hs1_ref[...]) + b1_ref[...]
    z = jnp.maximum(out1, 0.0)
    h2 = jnp.dot(z, w2_ref[...], preferred_element_type=jnp.float32)
    hs2_ref[...] = h2 * dinv


def _tc3_body(accp_ref, hs2_ref, dinv_ref, b2_ref, out_ref):
    logits = (dinv_ref[...] * (accp_ref[0] + accp_ref[1] + hs2_ref[...])
              + b2_ref[...])
    m = jnp.max(logits, axis=1, keepdims=True)
    lse = jnp.log(jnp.sum(jnp.exp(logits - m), axis=1, keepdims=True))
    out_ref[...] = logits - m - lse


_G = 5
_BN = N // _G

_tc1 = pl.pallas_call(
    _tc1_body,
    grid=(_G,),
    in_specs=[
        pl.BlockSpec((NC, _BN, 16), lambda i: (0, i, 0)),
        pl.BlockSpec((_BN, F_IN), lambda i: (i, 0)),
        pl.BlockSpec((F_IN, HID), lambda i: (0, 0)),
    ],
    out_specs=[
        pl.BlockSpec((_BN, 1), lambda i: (i, 0)),
        pl.BlockSpec((_BN, HID), lambda i: (i, 0)),
    ],
    out_shape=[
        jax.ShapeDtypeStruct((N, 1), jnp.float32),
        jax.ShapeDtypeStruct((N, HID), jnp.float32),
    ],
)

_tc2 = pl.pallas_call(
    _tc2_body,
    grid=(_G,),
    in_specs=[
        pl.BlockSpec((NC, _BN, HID), lambda i: (0, i, 0)),
        pl.BlockSpec((_BN, HID), lambda i: (i, 0)),
        pl.BlockSpec((_BN, 1), lambda i: (i, 0)),
        pl.BlockSpec((HID, CLS), lambda i: (0, 0)),
        pl.BlockSpec((1, HID), lambda i: (0, 0)),
    ],
    out_specs=pl.BlockSpec((_BN, CLS), lambda i: (i, 0)),
    out_shape=jax.ShapeDtypeStruct((N, CLS), jnp.float32),
)

_tc3 = pl.pallas_call(
    _tc3_body,
    grid=(_G,),
    in_specs=[
        pl.BlockSpec((NC, _BN, CLS), lambda i: (0, i, 0)),
        pl.BlockSpec((_BN, CLS), lambda i: (i, 0)),
        pl.BlockSpec((_BN, 1), lambda i: (i, 0)),
        pl.BlockSpec((1, CLS), lambda i: (0, 0)),
    ],
    out_specs=pl.BlockSpec((_BN, CLS), lambda i: (i, 0)),
    out_shape=jax.ShapeDtypeStruct((N, CLS), jnp.float32),
)


@jax.jit
def kernel(x, edge_index, W1, b1, W2, b2):
    srch = edge_index[0].reshape(NW, NCH, BH)
    dsth = edge_index[1].reshape(NW, NCH, BH)
    srcc = edge_index[0].reshape(NW, NCC, BC)
    dstc = edge_index[1].reshape(NW, NCC, BC)
    b1r = b1.reshape(1, HID)
    b2r = b2.reshape(1, CLS)

    degp = _deg_kernel(dstc)
    dinv, hs1 = _tc1(degp, x, W1)
    acc1 = _prop_hid(srch, dsth, hs1)
    hs2 = _tc2(acc1, hs1, dinv, W2, b1r)
    acc2 = _prop_cls(srcc, dstc, hs2)
    return _tc3(acc2, hs2, dinv, b2r)


# speedup vs baseline: 1.1465x; 1.1020x over previous
import functools

import jax
import jax.numpy as jnp
from jax import lax
from jax.experimental import pallas as pl
from jax.experimental.pallas import tpu as pltpu
from jax.experimental.pallas import tpu_sc as plsc

N = 10000
E = 320000
F_IN = 128
HID = 128
CLS = 16

NC = 2
NS = 16
NW = NC * NS
EPW = E // NW
RPT = N // NS
ZR = 25

BH = 125
NCH = EPW // BH
HALF = NCH // 2
BC = 2500
NCC = EPW // BC

_MESH = plsc.VectorSubcoreMesh(core_axis_name="c", subcore_axis_name="s")
_SC_PARAMS = pltpu.CompilerParams(use_tc_tiling_on_sc=False)


def _zero_fill(buf, nrows, ncol):
    def body(i, _):
        for k in range(ncol // 16):
            buf[i, pl.ds(k * 16, 16)] = jnp.zeros((16,), jnp.float32)
        return 0
    lax.fori_loop(0, nrows, body, 0)


@functools.partial(
    pl.kernel,
    out_type=jax.ShapeDtypeStruct((NC, N, 16), jnp.float32),
    mesh=_MESH,
    compiler_params=_SC_PARAMS,
    scratch_types=[
        pltpu.VMEM_SHARED((N, 16), jnp.float32),
        pltpu.VMEM((NCC, BC), jnp.int32),
        pltpu.VMEM((BC, 16), jnp.float32),
        pltpu.VMEM((RPT, 16), jnp.float32),
        pltpu.SemaphoreType.DMA,
    ],
)
def _deg_kernel(dst_hbm, out_hbm, acc, dstv, ones, zbuf, sem):
    c = lax.axis_index("c")
    s = lax.axis_index("s")
    wid = c * NS + s

    def fill_ones(i, _):
        for k in range(4):
            ones[i * 4 + k, :] = jnp.ones((16,), jnp.float32)
        return 0
    lax.fori_loop(0, BC // 4, fill_ones, 0)
    def fill_zero(i, _):
        for k in range(5):
            zbuf[i * 5 + k, :] = jnp.zeros((16,), jnp.float32)
        return 0
    lax.fori_loop(0, RPT // 5, fill_zero, 0)

    pltpu.sync_copy(dst_hbm.at[wid], dstv)
    pltpu.sync_copy(zbuf, acc.at[pl.ds(s * RPT, RPT)])
    plsc.subcore_barrier()

    def chunk(i, _):
        pltpu.async_copy(ones, acc.at[dstv.at[i]], sem, add=True)
        return 0
    lax.fori_loop(0, NCC, chunk, 0)
    def drain(i, _):
        pltpu.make_async_copy(ones, acc.at[dstv.at[i]], sem).wait()
        return 0
    lax.fori_loop(0, NCC, drain, 0)
    plsc.subcore_barrier()

    pltpu.sync_copy(acc.at[pl.ds(s * RPT, RPT)],
                    out_hbm.at[c].at[pl.ds(s * RPT, RPT)])


def _make_prop(F, CB, CN, PH):
    CPP = CN // PH

    @functools.partial(
        pl.kernel,
        out_type=jax.ShapeDtypeStruct((NC, N, F), jnp.float32),
        mesh=_MESH,
        compiler_params=_SC_PARAMS,
        scratch_types=[
            pltpu.VMEM_SHARED((N, F), jnp.float32),
            pltpu.VMEM((CPP, CB), jnp.int32),
            pltpu.VMEM((CPP, CB), jnp.int32),
            pltpu.VMEM((CB, F), jnp.float32),
            pltpu.VMEM((CB, F), jnp.float32),
            pltpu.VMEM((ZR, F), jnp.float32),
            pltpu.SemaphoreType.DMA,
            pltpu.SemaphoreType.DMA,
            pltpu.SemaphoreType.DMA,
            pltpu.SemaphoreType.DMA,
        ],
    )
    def _prop(src_hbm, dst_hbm, hs_hbm, out_hbm, acc, srcv, dstv,
              rows0, rows1, zbuf, semg0, semg1, sems0, sems1):
        c = lax.axis_index("c")
        s = lax.axis_index("s")
        wid = c * NS + s

        _zero_fill(zbuf, ZR, F)
        def zinit(r, _):
            pltpu.async_copy(zbuf, acc.at[pl.ds(s * RPT + r * ZR, ZR)], semg0)
            return 0
        lax.fori_loop(0, RPT // ZR, zinit, 0)
        def zdrain(r, _):
            pltpu.make_async_copy(
                zbuf, acc.at[pl.ds(s * RPT + r * ZR, ZR)], semg0).wait()
            return 0
        lax.fori_loop(0, RPT // ZR, zdrain, 0)

        for ph in range(PH):
            if PH == 1:
                pltpu.sync_copy(src_hbm.at[wid], srcv)
                pltpu.sync_copy(dst_hbm.at[wid], dstv)
            else:
                pltpu.sync_copy(src_hbm.at[wid].at[pl.ds(ph * CPP, CPP)], srcv)
                pltpu.sync_copy(dst_hbm.at[wid].at[pl.ds(ph * CPP, CPP)], dstv)
            if ph == 0:
                plsc.subcore_barrier()

            pltpu.async_copy(hs_hbm.at[srcv.at[0]], rows0, semg0)

            def pair(p, _):
                i = 2 * p
                pltpu.make_async_copy(hs_hbm.at[srcv.at[i]], rows0,
                                      semg0).wait()
                pltpu.async_copy(rows0, acc.at[dstv.at[i]], sems0, add=True)
                pltpu.async_copy(hs_hbm.at[srcv.at[i + 1]], rows1, semg1)
                pltpu.make_async_copy(hs_hbm.at[srcv.at[i + 1]], rows1,
                                      semg1).wait()
                pltpu.make_async_copy(rows0, acc.at[dstv.at[i]], sems0).wait()
                pltpu.async_copy(rows1, acc.at[dstv.at[i + 1]], sems1,
                                 add=True)

                @pl.when(i + 2 < CPP)
                def _():
                    pltpu.async_copy(hs_hbm.at[srcv.at[i + 2]], rows0, semg0)
                pltpu.make_async_copy(rows1, acc.at[dstv.at[i + 1]],
                                      sems1).wait()
                return 0
            lax.fori_loop(0, CPP // 2, pair, 0)

        plsc.subcore_barrier()
        pltpu.sync_copy(acc.at[pl.ds(s * RPT, RPT)],
                        out_hbm.at[c].at[pl.ds(s * RPT, RPT)])

    return _prop


B3 = 100
CN3 = EPW // B3
PH3 = 4
CPP3 = CN3 // PH3


@functools.partial(
    pl.kernel,
    out_type=jax.ShapeDtypeStruct((NC, N, HID), jnp.float32),
    mesh=_MESH,
    compiler_params=_SC_PARAMS,
    scratch_types=[
        pltpu.VMEM_SHARED((N, HID), jnp.float32),
        pltpu.VMEM((CPP3, B3), jnp.int32),
        pltpu.VMEM((CPP3, B3), jnp.int32),
        pltpu.VMEM((B3, HID), jnp.float32),
        pltpu.VMEM((B3, HID), jnp.float32),
        pltpu.VMEM((B3, HID), jnp.float32),
        pltpu.VMEM((ZR, HID), jnp.float32),
        pltpu.SemaphoreType.DMA,
        pltpu.SemaphoreType.DMA,
        pltpu.SemaphoreType.DMA,
        pltpu.SemaphoreType.DMA,
        pltpu.SemaphoreType.DMA,
        pltpu.SemaphoreType.DMA,
    ],
)
def _prop_hid(src_hbm, dst_hbm, hs_hbm, out_hbm, acc, srcv, dstv,
              r0, r1, r2, zbuf, g0, g1, g2, s0, s1, s2):
    c = lax.axis_index("c")
    s = lax.axis_index("s")
    wid = c * NS + s
    rows = [r0, r1, r2]
    gsem = [g0, g1, g2]
    ssem = [s0, s1, s2]

    _zero_fill(zbuf, ZR, HID)
    def zinit(r, _):
        pltpu.async_copy(zbuf, acc.at[pl.ds(s * RPT + r * ZR, ZR)], g0)
        return 0
    lax.fori_loop(0, RPT // ZR, zinit, 0)
    def zdrain(r, _):
        pltpu.make_async_copy(
            zbuf, acc.at[pl.ds(s * RPT + r * ZR, ZR)], g0).wait()
        return 0
    lax.fori_loop(0, RPT // ZR, zdrain, 0)

    for ph in range(PH3):
        pltpu.sync_copy(src_hbm.at[wid].at[pl.ds(ph * CPP3, CPP3)], srcv)
        pltpu.sync_copy(dst_hbm.at[wid].at[pl.ds(ph * CPP3, CPP3)], dstv)
        if ph == 0:
            plsc.subcore_barrier()

        pltpu.async_copy(hs_hbm.at[srcv.at[0]], r0, g0)
        pltpu.async_copy(hs_hbm.at[srcv.at[1]], r1, g1)

        def chunk_step(i, k):
            pltpu.make_async_copy(hs_hbm.at[srcv.at[i]], rows[k],
                                  gsem[k]).wait()
            pltpu.async_copy(rows[k], acc.at[dstv.at[i]], ssem[k], add=True)
            j = i + 2
            m = (k + 2) % 3

            @pl.when(j < CPP3)
            def _():
                @pl.when(i >= 1)
                def _():
                    pltpu.make_async_copy(rows[m], acc.at[dstv.at[i - 1]],
                                          ssem[m]).wait()
                pltpu.async_copy(hs_hbm.at[srcv.at[j]], rows[m], gsem[m])

        def triple(t, _):
            for k in range(3):
                chunk_step(3 * t + k, k)
            return 0
        lax.fori_loop(0, CPP3 // 3, triple, 0)
        for k in range(CPP3 - (CPP3 // 3) * 3):
            chunk_step((CPP3 // 3) * 3 + k, k)

        for back in range(3):
            i = CPP3 - 3 + back
            pltpu.make_async_copy(rows[i % 3], acc.at[dstv.at[i]],
                                  ssem[i % 3]).wait()

    plsc.subcore_barrier()
    pltpu.sync_copy(acc.at[pl.ds(s * RPT, RPT)],
                    out_hbm.at[c].at[pl.ds(s * RPT, RPT)])


_prop_cls = _make_prop(CLS, BC, NCC, 1)


def _tc1_body(degp_ref, x_ref, w1_ref, dinv_ref, hs1_ref):
    deg = (jnp.max(degp_ref[0], axis=1, keepdims=True)
           + jnp.max(degp_ref[1], axis=1, keepdims=True) + 1.0)
    dinv = lax.rsqrt(deg)
    dinv_ref[...] = dinv
    h = jnp.dot(x_ref[...], w1_ref[...], preferred_element_type=jnp.float32)
    hs1_ref[...] = h * dinv


def _tc2_body(accp_ref, hs1_ref, dinv_ref, w2_ref, b1_ref, hs2_ref):
    dinv = dinv_ref[...]
    out1 = dinv * (accp_ref[0] + accp_ref[1] + hs1_ref[...]) + b1_ref[...]
    z = jnp.maximum(out1, 0.0)
    h2 = jnp.dot(z, w2_ref[...], preferred_element_type=jnp.float32)
    hs2_ref[...] = h2 * dinv


def _tc3_body(accp_ref, hs2_ref, dinv_ref, b2_ref, out_ref):
    logits = (dinv_ref[...] * (accp_ref[0] + accp_ref[1] + hs2_ref[...])
              + b2_ref[...])
    m = jnp.max(logits, axis=1, keepdims=True)
    lse = jnp.log(jnp.sum(jnp.exp(logits - m), axis=1, keepdims=True))
    out_ref[...] = logits - m - lse


_G = 5
_BN = N // _G

_tc1 = pl.pallas_call(
    _tc1_body,
    grid=(_G,),
    in_specs=[
        pl.BlockSpec((NC, _BN, 16), lambda i: (0, i, 0)),
        pl.BlockSpec((_BN, F_IN), lambda i: (i, 0)),
        pl.BlockSpec((F_IN, HID), lambda i: (0, 0)),
    ],
    out_specs=[
        pl.BlockSpec((_BN, 1), lambda i: (i, 0)),
        pl.BlockSpec((_BN, HID), lambda i: (i, 0)),
    ],
    out_shape=[
        jax.ShapeDtypeStruct((N, 1), jnp.float32),
        jax.ShapeDtypeStruct((N, HID), jnp.float32),
    ],
)

_tc2 = pl.pallas_call(
    _tc2_body,
    grid=(_G,),
    in_specs=[
        pl.BlockSpec((NC, _BN, HID), lambda i: (0, i, 0)),
        pl.BlockSpec((_BN, HID), lambda i: (i, 0)),
        pl.BlockSpec((_BN, 1), lambda i: (i, 0)),
        pl.BlockSpec((HID, CLS), lambda i: (0, 0)),
        pl.BlockSpec((1, HID), lambda i: (0, 0)),
    ],
    out_specs=pl.BlockSpec((_BN, CLS), lambda i: (i, 0)),
    out_shape=jax.ShapeDtypeStruct((N, CLS), jnp.float32),
)

_tc3 = pl.pallas_call(
    _tc3_body,
    grid=(_G,),
    in_specs=[
        pl.BlockSpec((NC, _BN, CLS), lambda i: (0, i, 0)),
        pl.BlockSpec((_BN, CLS), lambda i: (i, 0)),
        pl.BlockSpec((_BN, 1), lambda i: (i, 0)),
        pl.BlockSpec((1, CLS), lambda i: (0, 0)),
    ],
    out_specs=pl.BlockSpec((_BN, CLS), lambda i: (i, 0)),
    out_shape=jax.ShapeDtypeStruct((N, CLS), jnp.float32),
)


@jax.jit
def kernel(x, edge_index, W1, b1, W2, b2):
    srch = edge_index[0].reshape(NW, CN3, B3)
    dsth = edge_index[1].reshape(NW, CN3, B3)
    srcc = edge_index[0].reshape(NW, NCC, BC)
    dstc = edge_index[1].reshape(NW, NCC, BC)
    b1r = b1.reshape(1, HID)
    b2r = b2.reshape(1, CLS)

    degp = _deg_kernel(dstc)
    dinv, hs1 = _tc1(degp, x, W1)
    acc1 = _prop_hid(srch, dsth, hs1)
    hs2 = _tc2(acc1, hs1, dinv, W2, b1r)
    acc2 = _prop_cls(srcc, dstc, hs2)
    return _tc3(acc2, hs2, dinv, b2r)


# speedup vs baseline: 1.1640x; 1.0152x over previous
import functools

import jax
import jax.numpy as jnp
from jax import lax
from jax.experimental import pallas as pl
from jax.experimental.pallas import tpu as pltpu
from jax.experimental.pallas import tpu_sc as plsc

N = 10000
E = 320000
F_IN = 128
HID = 128
CLS = 16

NC = 2
NS = 16
NW = NC * NS
EPW = E // NW
RPT = N // NS
ZR = 25

BH = 125
NCH = EPW // BH
HALF = NCH // 2
BC = 2500
NCC = EPW // BC

_MESH = plsc.VectorSubcoreMesh(core_axis_name="c", subcore_axis_name="s")
_SC_PARAMS = pltpu.CompilerParams(use_tc_tiling_on_sc=False)


def _zero_fill(buf, nrows, ncol):
    def body(i, _):
        for k in range(ncol // 16):
            buf[i, pl.ds(k * 16, 16)] = jnp.zeros((16,), jnp.float32)
        return 0
    lax.fori_loop(0, nrows, body, 0)


@functools.partial(
    pl.kernel,
    out_type=jax.ShapeDtypeStruct((NC, N, 16), jnp.float32),
    mesh=_MESH,
    compiler_params=_SC_PARAMS,
    scratch_types=[
        pltpu.VMEM_SHARED((N, 16), jnp.float32),
        pltpu.VMEM((NCC, BC), jnp.int32),
        pltpu.VMEM((BC, 16), jnp.float32),
        pltpu.VMEM((RPT, 16), jnp.float32),
        pltpu.SemaphoreType.DMA,
    ],
)
def _deg_kernel(dst_hbm, out_hbm, acc, dstv, ones, zbuf, sem):
    c = lax.axis_index("c")
    s = lax.axis_index("s")
    wid = c * NS + s

    def fill_ones(i, _):
        for k in range(4):
            ones[i * 4 + k, :] = jnp.ones((16,), jnp.float32)
        return 0
    lax.fori_loop(0, BC // 4, fill_ones, 0)
    def fill_zero(i, _):
        for k in range(5):
            zbuf[i * 5 + k, :] = jnp.zeros((16,), jnp.float32)
        return 0
    lax.fori_loop(0, RPT // 5, fill_zero, 0)

    pltpu.sync_copy(dst_hbm.at[wid], dstv)
    pltpu.sync_copy(zbuf, acc.at[pl.ds(s * RPT, RPT)])
    plsc.subcore_barrier()

    def chunk(i, _):
        pltpu.async_copy(ones, acc.at[dstv.at[i]], sem, add=True)
        return 0
    lax.fori_loop(0, NCC, chunk, 0)
    def drain(i, _):
        pltpu.make_async_copy(ones, acc.at[dstv.at[i]], sem).wait()
        return 0
    lax.fori_loop(0, NCC, drain, 0)
    plsc.subcore_barrier()

    pltpu.sync_copy(acc.at[pl.ds(s * RPT, RPT)],
                    out_hbm.at[c].at[pl.ds(s * RPT, RPT)])


def _make_prop(F, CB, CN, PH):
    CPP = CN // PH

    @functools.partial(
        pl.kernel,
        out_type=jax.ShapeDtypeStruct((NC, N, F), jnp.float32),
        mesh=_MESH,
        compiler_params=_SC_PARAMS,
        scratch_types=[
            pltpu.VMEM_SHARED((N, F), jnp.float32),
            pltpu.VMEM((CPP, CB), jnp.int32),
            pltpu.VMEM((CPP, CB), jnp.int32),
            pltpu.VMEM((CB, F), jnp.float32),
            pltpu.VMEM((CB, F), jnp.float32),
            pltpu.VMEM((ZR, F), jnp.float32),
            pltpu.SemaphoreType.DMA,
            pltpu.SemaphoreType.DMA,
            pltpu.SemaphoreType.DMA,
            pltpu.SemaphoreType.DMA,
        ],
    )
    def _prop(src_hbm, dst_hbm, hs_hbm, out_hbm, acc, srcv, dstv,
              rows0, rows1, zbuf, semg0, semg1, sems0, sems1):
        c = lax.axis_index("c")
        s = lax.axis_index("s")
        wid = c * NS + s

        _zero_fill(zbuf, ZR, F)
        def zinit(r, _):
            pltpu.async_copy(zbuf, acc.at[pl.ds(s * RPT + r * ZR, ZR)], semg0)
            return 0
        lax.fori_loop(0, RPT // ZR, zinit, 0)
        def zdrain(r, _):
            pltpu.make_async_copy(
                zbuf, acc.at[pl.ds(s * RPT + r * ZR, ZR)], semg0).wait()
            return 0
        lax.fori_loop(0, RPT // ZR, zdrain, 0)

        for ph in range(PH):
            if PH == 1:
                pltpu.sync_copy(src_hbm.at[wid], srcv)
                pltpu.sync_copy(dst_hbm.at[wid], dstv)
            else:
                pltpu.sync_copy(src_hbm.at[wid].at[pl.ds(ph * CPP, CPP)], srcv)
                pltpu.sync_copy(dst_hbm.at[wid].at[pl.ds(ph * CPP, CPP)], dstv)
            if ph == 0:
                plsc.subcore_barrier()

            pltpu.async_copy(hs_hbm.at[srcv.at[0]], rows0, semg0)

            def pair(p, _):
                i = 2 * p
                pltpu.make_async_copy(hs_hbm.at[srcv.at[i]], rows0,
                                      semg0).wait()
                pltpu.async_copy(rows0, acc.at[dstv.at[i]], sems0, add=True)
                pltpu.async_copy(hs_hbm.at[srcv.at[i + 1]], rows1, semg1)
                pltpu.make_async_copy(hs_hbm.at[srcv.at[i + 1]], rows1,
                                      semg1).wait()
                pltpu.make_async_copy(rows0, acc.at[dstv.at[i]], sems0).wait()
                pltpu.async_copy(rows1, acc.at[dstv.at[i + 1]], sems1,
                                 add=True)

                @pl.when(i + 2 < CPP)
                def _():
                    pltpu.async_copy(hs_hbm.at[srcv.at[i + 2]], rows0, semg0)
                pltpu.make_async_copy(rows1, acc.at[dstv.at[i + 1]],
                                      sems1).wait()
                return 0
            lax.fori_loop(0, CPP // 2, pair, 0)

        plsc.subcore_barrier()
        pltpu.sync_copy(acc.at[pl.ds(s * RPT, RPT)],
                        out_hbm.at[c].at[pl.ds(s * RPT, RPT)])

    return _prop


B3 = 100
CN3 = EPW // B3
PH3 = 4
CPP3 = CN3 // PH3


@functools.partial(
    pl.kernel,
    out_type=jax.ShapeDtypeStruct((NC, N, HID), jnp.float32),
    mesh=_MESH,
    compiler_params=_SC_PARAMS,
    scratch_types=[
        pltpu.VMEM_SHARED((N, HID), jnp.float32),
        pltpu.VMEM((CPP3, B3), jnp.int32),
        pltpu.VMEM((CPP3, B3), jnp.int32),
        pltpu.VMEM((B3, HID), jnp.float32),
        pltpu.VMEM((B3, HID), jnp.float32),
        pltpu.VMEM((B3, HID), jnp.float32),
        pltpu.VMEM((ZR, HID), jnp.float32),
        pltpu.SemaphoreType.DMA,
        pltpu.SemaphoreType.DMA,
        pltpu.SemaphoreType.DMA,
        pltpu.SemaphoreType.DMA,
        pltpu.SemaphoreType.DMA,
        pltpu.SemaphoreType.DMA,
    ],
)
def _prop_hid(src_hbm, dst_hbm, hs_hbm, out_hbm, acc, srcv, dstv,
              r0, r1, r2, zbuf, g0, g1, g2, s0, s1, s2):
    c = lax.axis_index("c")
    s = lax.axis_index("s")
    wid = c * NS + s
    rows = [r0, r1, r2]
    gsem = [g0, g1, g2]
    ssem = [s0, s1, s2]

    _zero_fill(zbuf, ZR, HID)
    def zinit(r, _):
        pltpu.async_copy(zbuf, acc.at[pl.ds(s * RPT + r * ZR, ZR)], g0)
        return 0
    lax.fori_loop(0, RPT // ZR, zinit, 0)
    def zdrain(r, _):
        pltpu.make_async_copy(
            zbuf, acc.at[pl.ds(s * RPT + r * ZR, ZR)], g0).wait()
        return 0
    lax.fori_loop(0, RPT // ZR, zdrain, 0)

    for ph in range(PH3):
        pltpu.sync_copy(src_hbm.at[wid].at[pl.ds(ph * CPP3, CPP3)], srcv)
        pltpu.sync_copy(dst_hbm.at[wid].at[pl.ds(ph * CPP3, CPP3)], dstv)
        if ph == 0:
            plsc.subcore_barrier()

        pltpu.async_copy(hs_hbm.at[srcv.at[0]], r0, g0)
        pltpu.async_copy(hs_hbm.at[srcv.at[1]], r1, g1)

        def chunk_step(i, k):
            pltpu.make_async_copy(hs_hbm.at[srcv.at[i]], rows[k],
                                  gsem[k]).wait()
            pltpu.async_copy(rows[k], acc.at[dstv.at[i]], ssem[k], add=True)
            j = i + 2
            m = (k + 2) % 3

            @pl.when(j < CPP3)
            def _():
                @pl.when(i >= 1)
                def _():
                    pltpu.make_async_copy(rows[m], acc.at[dstv.at[i - 1]],
                                          ssem[m]).wait()
                pltpu.async_copy(hs_hbm.at[srcv.at[j]], rows[m], gsem[m])

        def triple(t, _):
            for k in range(3):
                chunk_step(3 * t + k, k)
            return 0
        lax.fori_loop(0, CPP3 // 3, triple, 0)
        for k in range(CPP3 - (CPP3 // 3) * 3):
            chunk_step((CPP3 // 3) * 3 + k, k)

        for back in range(3):
            i = CPP3 - 3 + back
            pltpu.make_async_copy(rows[i % 3], acc.at[dstv.at[i]],
                                  ssem[i % 3]).wait()

    plsc.subcore_barrier()
    pltpu.sync_copy(acc.at[pl.ds(s * RPT, RPT)],
                    out_hbm.at[c].at[pl.ds(s * RPT, RPT)])


_prop_cls = _make_prop(CLS, BC, NCC, 1)


def _tc1_body(degp_ref, x_ref, w1_ref, dinv_ref, hs1_ref):
    deg = (jnp.max(degp_ref[0], axis=1, keepdims=True)
           + jnp.max(degp_ref[1], axis=1, keepdims=True) + 1.0)
    dinv = lax.rsqrt(deg)
    dinv_ref[...] = dinv
    h = jnp.dot(x_ref[...], w1_ref[...], preferred_element_type=jnp.float32)
    hs1_ref[...] = h * dinv


def _tc2_body(accp_ref, hs1_ref, dinv_ref, w2_ref, b1_ref, hs2_ref):
    dinv = dinv_ref[...]
    out1 = dinv * (accp_ref[0] + accp_ref[1] + hs1_ref[...]) + b1_ref[...]
    z = jnp.maximum(out1, 0.0)
    h2 = jnp.dot(z, w2_ref[...], preferred_element_type=jnp.float32)
    hs2_ref[...] = h2 * dinv


def _tc3_body(accp_ref, hs2_ref, dinv_ref, b2_ref, out_ref):
    logits = (dinv_ref[...] * (accp_ref[0] + accp_ref[1] + hs2_ref[...])
              + b2_ref[...])
    m = jnp.max(logits, axis=1, keepdims=True)
    lse = jnp.log(jnp.sum(jnp.exp(logits - m), axis=1, keepdims=True))
    out_ref[...] = logits - m - lse


_G = 2
_BN = N // _G

_tc1 = pl.pallas_call(
    _tc1_body,
    grid=(_G,),
    in_specs=[
        pl.BlockSpec((NC, _BN, 16), lambda i: (0, i, 0)),
        pl.BlockSpec((_BN, F_IN), lambda i: (i, 0)),
        pl.BlockSpec((F_IN, HID), lambda i: (0, 0)),
    ],
    out_specs=[
        pl.BlockSpec((_BN, 1), lambda i: (i, 0)),
        pl.BlockSpec((_BN, HID), lambda i: (i, 0)),
    ],
    out_shape=[
        jax.ShapeDtypeStruct((N, 1), jnp.float32),
        jax.ShapeDtypeStruct((N, HID), jnp.float32),
    ],
)

_tc2 = pl.pallas_call(
    _tc2_body,
    grid=(_G,),
    in_specs=[
        pl.BlockSpec((NC, _BN, HID), lambda i: (0, i, 0)),
        pl.BlockSpec((_BN, HID), lambda i: (i, 0)),
        pl.BlockSpec((_BN, 1), lambda i: (i, 0)),
        pl.BlockSpec((HID, CLS), lambda i: (0, 0)),
        pl.BlockSpec((1, HID), lambda i: (0, 0)),
    ],
    out_specs=pl.BlockSpec((_BN, CLS), lambda i: (i, 0)),
    out_shape=jax.ShapeDtypeStruct((N, CLS), jnp.float32),
)

_tc3 = pl.pallas_call(
    _tc3_body,
    grid=(_G,),
    in_specs=[
        pl.BlockSpec((NC, _BN, CLS), lambda i: (0, i, 0)),
        pl.BlockSpec((_BN, CLS), lambda i: (i, 0)),
        pl.BlockSpec((_BN, 1), lambda i: (i, 0)),
        pl.BlockSpec((1, CLS), lambda i: (0, 0)),
    ],
    out_specs=pl.BlockSpec((_BN, CLS), lambda i: (i, 0)),
    out_shape=jax.ShapeDtypeStruct((N, CLS), jnp.float32),
)


@jax.jit
def kernel(x, edge_index, W1, b1, W2, b2):
    srch = edge_index[0].reshape(NW, CN3, B3)
    dsth = edge_index[1].reshape(NW, CN3, B3)
    srcc = edge_index[0].reshape(NW, NCC, BC)
    dstc = edge_index[1].reshape(NW, NCC, BC)
    b1r = b1.reshape(1, HID)
    b2r = b2.reshape(1, CLS)

    degp = _deg_kernel(dstc)
    dinv, hs1 = _tc1(degp, x, W1)
    acc1 = _prop_hid(srch, dsth, hs1)
    hs2 = _tc2(acc1, hs1, dinv, W2, b1r)
    acc2 = _prop_cls(srcc, dstc, hs2)
    return _tc3(acc2, hs2, dinv, b2r)


# speedup vs baseline: 1.1782x; 1.0122x over previous
import functools

import jax
import jax.numpy as jnp
from jax import lax
from jax.experimental import pallas as pl
from jax.experimental.pallas import tpu as pltpu
from jax.experimental.pallas import tpu_sc as plsc

N = 10000
E = 320000
F_IN = 128
HID = 128
CLS = 16

NC = 2
NS = 16
NW = NC * NS
EPW = E // NW
RPT = N // NS
ZR = 25

BH = 125
NCH = EPW // BH
HALF = NCH // 2
BC = 2000
NCC = EPW // BC
B3 = 100
CN3 = EPW // B3
PH3 = 4
CPP3 = CN3 // PH3

_MESH = plsc.VectorSubcoreMesh(core_axis_name="c", subcore_axis_name="s")
_SC_PARAMS = pltpu.CompilerParams(use_tc_tiling_on_sc=False)


def _zero_fill(buf, nrows, ncol):
    def body(i, _):
        for k in range(ncol // 16):
            buf[i, pl.ds(k * 16, 16)] = jnp.zeros((16,), jnp.float32)
        return 0
    lax.fori_loop(0, nrows, body, 0)


@functools.partial(
    pl.kernel,
    out_type=jax.ShapeDtypeStruct((NC, N, 16), jnp.float32),
    mesh=_MESH,
    compiler_params=_SC_PARAMS,
    scratch_types=[
        pltpu.VMEM_SHARED((N, 16), jnp.float32),
        pltpu.VMEM((NCC, BC), jnp.int32),
        pltpu.VMEM((BC, 16), jnp.float32),
        pltpu.VMEM((RPT, 16), jnp.float32),
        pltpu.SemaphoreType.DMA,
    ],
)
def _deg_kernel(dst_hbm, out_hbm, acc, dstv, ones, zbuf, sem):
    c = lax.axis_index("c")
    s = lax.axis_index("s")
    wid = c * NS + s

    def fill_ones(i, _):
        for k in range(4):
            ones[i * 4 + k, :] = jnp.ones((16,), jnp.float32)
        return 0
    lax.fori_loop(0, BC // 4, fill_ones, 0)
    def fill_zero(i, _):
        for k in range(5):
            zbuf[i * 5 + k, :] = jnp.zeros((16,), jnp.float32)
        return 0
    lax.fori_loop(0, RPT // 5, fill_zero, 0)

    pltpu.sync_copy(dst_hbm.at[wid], dstv)
    pltpu.sync_copy(zbuf, acc.at[pl.ds(s * RPT, RPT)])
    plsc.subcore_barrier()

    def chunk(i, _):
        pltpu.async_copy(ones, acc.at[dstv.at[i]], sem, add=True)
        return 0
    lax.fori_loop(0, NCC, chunk, 0)
    def drain(i, _):
        pltpu.make_async_copy(ones, acc.at[dstv.at[i]], sem).wait()
        return 0
    lax.fori_loop(0, NCC, drain, 0)
    plsc.subcore_barrier()

    pltpu.sync_copy(acc.at[pl.ds(s * RPT, RPT)],
                    out_hbm.at[c].at[pl.ds(s * RPT, RPT)])


def _make_prop(F, CB, CN, PH):
    CPP = CN // PH

    @functools.partial(
        pl.kernel,
        out_type=jax.ShapeDtypeStruct((NC, N, F), jnp.float32),
        mesh=_MESH,
        compiler_params=_SC_PARAMS,
        scratch_types=[
            pltpu.VMEM_SHARED((N, F), jnp.float32),
            pltpu.VMEM((CPP, CB), jnp.int32),
            pltpu.VMEM((CPP, CB), jnp.int32),
            pltpu.VMEM((CB, F), jnp.float32),
            pltpu.VMEM((CB, F), jnp.float32),
            pltpu.VMEM((CB, F), jnp.float32),
            pltpu.VMEM((ZR, F), jnp.float32),
            pltpu.SemaphoreType.DMA,
            pltpu.SemaphoreType.DMA,
            pltpu.SemaphoreType.DMA,
            pltpu.SemaphoreType.DMA,
            pltpu.SemaphoreType.DMA,
            pltpu.SemaphoreType.DMA,
        ],
    )
    def _prop(src_hbm, dst_hbm, hs_hbm, out_hbm, acc, srcv, dstv,
              r0, r1, r2, zbuf, g0, g1, g2, s0, s1, s2):
        c = lax.axis_index("c")
        s = lax.axis_index("s")
        wid = c * NS + s
        rows = [r0, r1, r2]
        gsem = [g0, g1, g2]
        ssem = [s0, s1, s2]

        _zero_fill(zbuf, ZR, F)
        def zinit(r, _):
            pltpu.async_copy(zbuf, acc.at[pl.ds(s * RPT + r * ZR, ZR)], g0)
            return 0
        lax.fori_loop(0, RPT // ZR, zinit, 0)
        def zdrain(r, _):
            pltpu.make_async_copy(
                zbuf, acc.at[pl.ds(s * RPT + r * ZR, ZR)], g0).wait()
            return 0
        lax.fori_loop(0, RPT // ZR, zdrain, 0)

        for ph in range(PH):
            if PH == 1:
                pltpu.sync_copy(src_hbm.at[wid], srcv)
                pltpu.sync_copy(dst_hbm.at[wid], dstv)
            else:
                pltpu.sync_copy(src_hbm.at[wid].at[pl.ds(ph * CPP, CPP)],
                                srcv)
                pltpu.sync_copy(dst_hbm.at[wid].at[pl.ds(ph * CPP, CPP)],
                                dstv)
            if ph == 0:
                plsc.subcore_barrier()

            pltpu.async_copy(hs_hbm.at[srcv.at[0]], r0, g0)
            pltpu.async_copy(hs_hbm.at[srcv.at[1]], r1, g1)

            def chunk_step(i, k):
                pltpu.make_async_copy(hs_hbm.at[srcv.at[i]], rows[k],
                                      gsem[k]).wait()
                pltpu.async_copy(rows[k], acc.at[dstv.at[i]], ssem[k],
                                 add=True)
                j = i + 2
                m = (k + 2) % 3

                @pl.when(j < CPP)
                def _():
                    @pl.when(i >= 1)
                    def _():
                        pltpu.make_async_copy(rows[m],
                                              acc.at[dstv.at[i - 1]],
                                              ssem[m]).wait()
                    pltpu.async_copy(hs_hbm.at[srcv.at[j]], rows[m], gsem[m])

            def triple(t, _):
                for k in range(3):
                    chunk_step(3 * t + k, k)
                return 0
            lax.fori_loop(0, CPP // 3, triple, 0)
            for k in range(CPP - (CPP // 3) * 3):
                chunk_step((CPP // 3) * 3 + k, k)

            for back in range(3):
                i = CPP - 3 + back
                pltpu.make_async_copy(rows[i % 3], acc.at[dstv.at[i]],
                                      ssem[i % 3]).wait()

        plsc.subcore_barrier()
        pltpu.sync_copy(acc.at[pl.ds(s * RPT, RPT)],
                        out_hbm.at[c].at[pl.ds(s * RPT, RPT)])

    return _prop


_prop_hid = _make_prop(HID, B3, CN3, PH3)
_prop_cls = _make_prop(CLS, BC, NCC, 1)


def _tc1_body(degp_ref, x_ref, w1_ref, dinv_ref, hs1_ref):
    deg = (jnp.max(degp_ref[0], axis=1, keepdims=True)
           + jnp.max(degp_ref[1], axis=1, keepdims=True) + 1.0)
    dinv = lax.rsqrt(deg)
    dinv_ref[...] = dinv
    h = jnp.dot(x_ref[...], w1_ref[...], preferred_element_type=jnp.float32)
    hs1_ref[...] = h * dinv


def _tc2_body(accp_ref, hs1_ref, dinv_ref, w2_ref, b1_ref, hs2_ref):
    dinv = dinv_ref[...]
    out1 = dinv * (accp_ref[0] + accp_ref[1] + hs1_ref[...]) + b1_ref[...]
    z = jnp.maximum(out1, 0.0)
    h2 = jnp.dot(z, w2_ref[...], preferred_element_type=jnp.float32)
    hs2_ref[...] = h2 * dinv


def _tc3_body(accp_ref, hs2_ref, dinv_ref, b2_ref, out_ref):
    logits = (dinv_ref[...] * (accp_ref[0] + accp_ref[1] + hs2_ref[...])
              + b2_ref[...])
    m = jnp.max(logits, axis=1, keepdims=True)
    lse = jnp.log(jnp.sum(jnp.exp(logits - m), axis=1, keepdims=True))
    out_ref[...] = logits - m - lse


_G = 2
_BN = N // _G

_tc1 = pl.pallas_call(
    _tc1_body,
    grid=(_G,),
    in_specs=[
        pl.BlockSpec((NC, _BN, 16), lambda i: (0, i, 0)),
        pl.BlockSpec((_BN, F_IN), lambda i: (i, 0)),
        pl.BlockSpec((F_IN, HID), lambda i: (0, 0)),
    ],
    out_specs=[
        pl.BlockSpec((_BN, 1), lambda i: (i, 0)),
        pl.BlockSpec((_BN, HID), lambda i: (i, 0)),
    ],
    out_shape=[
        jax.ShapeDtypeStruct((N, 1), jnp.float32),
        jax.ShapeDtypeStruct((N, HID), jnp.float32),
    ],
)

_tc2 = pl.pallas_call(
    _tc2_body,
    grid=(_G,),
    in_specs=[
        pl.BlockSpec((NC, _BN, HID), lambda i: (0, i, 0)),
        pl.BlockSpec((_BN, HID), lambda i: (i, 0)),
        pl.BlockSpec((_BN, 1), lambda i: (i, 0)),
        pl.BlockSpec((HID, CLS), lambda i: (0, 0)),
        pl.BlockSpec((1, HID), lambda i: (0, 0)),
    ],
    out_specs=pl.BlockSpec((_BN, CLS), lambda i: (i, 0)),
    out_shape=jax.ShapeDtypeStruct((N, CLS), jnp.float32),
)

_tc3 = pl.pallas_call(
    _tc3_body,
    grid=(_G,),
    in_specs=[
        pl.BlockSpec((NC, _BN, CLS), lambda i: (0, i, 0)),
        pl.BlockSpec((_BN, CLS), lambda i: (i, 0)),
        pl.BlockSpec((_BN, 1), lambda i: (i, 0)),
        pl.BlockSpec((1, CLS), lambda i: (0, 0)),
    ],
    out_specs=pl.BlockSpec((_BN, CLS), lambda i: (i, 0)),
    out_shape=jax.ShapeDtypeStruct((N, CLS), jnp.float32),
)


@jax.jit
def kernel(x, edge_index, W1, b1, W2, b2):
    srch = edge_index[0].reshape(NW, CN3, B3)
    dsth = edge_index[1].reshape(NW, CN3, B3)
    srcc = edge_index[0].reshape(NW, NCC, BC)
    dstc = edge_index[1].reshape(NW, NCC, BC)
    b1r = b1.reshape(1, HID)
    b2r = b2.reshape(1, CLS)

    degp = _deg_kernel(dstc)
    dinv, hs1 = _tc1(degp, x, W1)
    acc1 = _prop_hid(srch, dsth, hs1)
    hs2 = _tc2(acc1, hs1, dinv, W2, b1r)
    acc2 = _prop_cls(srcc, dstc, hs2)
    return _tc3(acc2, hs2, dinv, b2r)
